# baseline (speedup 1.0000x reference)
import math

import jax
import jax.numpy as jnp
from jax.experimental import pallas as pl
from jax.experimental.pallas import tpu as pltpu

_EMB = 96
_HEADS = 4
_DH = _EMB // _HEADS
_DEPTH = 6
_E = 8
_PATCH = 16
_IMG = 224
_NCLS = 1000
_HID = 4 * _EMB
_B = 128
_NP = (_IMG // _PATCH) ** 2
_T = _NP + 1
_NMOE = _DEPTH - _DEPTH // 2
_CAP = int(1.25 * _B * _T / _E)
_PD = 3 * _PATCH * _PATCH

_TP = 200
_G = 8
_R = _G * _TP


def _ln(v, w, b):
    mu = jnp.mean(v, axis=-1, keepdims=True)
    var = jnp.mean((v - mu) ** 2, axis=-1, keepdims=True)
    return (v - mu) * jax.lax.rsqrt(var + 1e-5) * w + b


def _vit_body(xp_ref, wp_ref, posf_ref,
              ln1w_ref, ln1b_ref, wq_ref, qb_ref, wk_ref, kb_ref,
              wv_ref, vb_ref, wo_ref, ob_ref, ln2w_ref, ln2b_ref,
              f1w_ref, f1b_ref, f2w_ref, f2b_ref, wr_ref, br_ref,
              nw_ref, nb_ref, hw_ref, hb_ref,
              out_ref, cnt_ref, ssq_ref):
    xp = xp_ref[...].reshape(_R, _PD)
    h = jnp.dot(xp, wp_ref[...],
                preferred_element_type=jnp.float32) + posf_ref[...]

    cid = jax.lax.broadcasted_iota(jnp.int32, (1, _R), 1)
    rmaskT = (cid % _TP) < _T
    cmask = jax.lax.broadcasted_iota(jnp.int32, (_TP, _TP), 1) < _T

    cnts = []
    ssqs = []
    for i in range(_DEPTH):
        hn = _ln(h, ln1w_ref[i], ln1b_ref[i])
        attn = jnp.zeros((1, _EMB), jnp.float32) + ob_ref[i]
        for hd in range(_HEADS):
            q = jnp.dot(hn, wq_ref[i, hd],
                        preferred_element_type=jnp.float32) + qb_ref[i, hd]
            k = jnp.dot(hn, wk_ref[i, hd],
                        preferred_element_type=jnp.float32) + kb_ref[i, hd]
            v = jnp.dot(hn, wv_ref[i, hd],
                        preferred_element_type=jnp.float32) + vb_ref[i, hd]
            os_ = []
            for g in range(_G):
                qg = q[g * _TP:(g + 1) * _TP]
                kg = k[g * _TP:(g + 1) * _TP]
                vg = v[g * _TP:(g + 1) * _TP]
                s = jax.lax.dot_general(
                    qg, kg, (((1,), (1,)), ((), ())),
                    preferred_element_type=jnp.float32)
                s = jnp.where(cmask, s, -1e30)
                m = jnp.max(s, axis=1, keepdims=True)
                e = jnp.exp(s - m)
                oa = jnp.dot(e, vg, preferred_element_type=jnp.float32)
                og = oa[:, :_DH] * (1.0 / oa[:, _DH:_DH + 1])
                os_.append(og)
            o = jnp.concatenate(os_, axis=0)
            attn = attn + jnp.dot(o, wo_ref[i, hd],
                                  preferred_element_type=jnp.float32)
        h = h + attn
        hn2 = _ln(h, ln2w_ref[i], ln2b_ref[i])
        if i < _DEPTH // 2:
            a = jnp.dot(hn2, f1w_ref[i],
                        preferred_element_type=jnp.float32) + f1b_ref[i]
            f = 0.5 * a * (1.0 + jax.lax.erf(a * (1.0 / math.sqrt(2.0))))
            h = h + jnp.dot(f, f2w_ref[i],
                            preferred_element_type=jnp.float32) + f2b_ref[i]
        else:
            j = i - _DEPTH // 2
            hn2T = hn2.T
            lgT = jnp.dot(wr_ref[j], hn2T,
                          preferred_element_type=jnp.float32) + br_ref[j]
            lgmT = jnp.where(rmaskT, lgT, 0.0)
            ssqs.append(jnp.sum(lgmT * lgmT).reshape(1, 1))
            ids = jax.lax.broadcasted_iota(jnp.int32, (_E, _R), 0)
            m1 = jnp.max(lgT, axis=0, keepdims=True)
            i1 = jnp.min(jnp.where(lgT == m1, ids, _E), axis=0, keepdims=True)
            lg2 = jnp.where(ids == i1, -jnp.inf, lgT)
            m2 = jnp.max(lg2, axis=0, keepdims=True)
            i2 = jnp.min(jnp.where(lg2 == m2, ids, _E), axis=0, keepdims=True)
            pres = jnp.logical_or(ids == i1, ids == i2) & rmaskT
            cnts.append(jnp.sum(pres.astype(jnp.float32), axis=1,
                                keepdims=True))

    cl = jnp.concatenate([h[g * _TP:g * _TP + 1] for g in range(_G)], axis=0)
    cls = _ln(cl, nw_ref[...], nb_ref[...])
    lgt = jnp.dot(cls, hw_ref[...],
                  preferred_element_type=jnp.float32) + hb_ref[...]
    out_ref[...] = lgt.reshape(_G, 1, _NCLS)
    cnt_ref[0] = jnp.concatenate(cnts, axis=1).T
    ssq_ref[0] = jnp.concatenate(ssqs, axis=1)


def _aux_body(cnt_ref, ssq_ref, out_ref):
    c = jnp.sum(cnt_ref[...], axis=0)
    usage = jnp.minimum(c, float(_CAP))
    load = usage / (jnp.sum(usage, axis=1, keepdims=True) + 1e-6)
    bal = float(_E) * jnp.sum(load * load, axis=1)
    ssq = jnp.sum(ssq_ref[...], axis=(0, 1))
    z = 0.001 * ssq / float(_B * _T * _E)
    out_ref[...] = jnp.sum(bal + z).reshape(1, 1)


def _full(shape):
    n = len(shape)
    return pl.BlockSpec(shape, lambda b: (0,) * n)


def kernel(x, params):
    p = params
    hO = _IMG // _PATCH
    xp = x.reshape(_B, 3, hO, _PATCH, hO, _PATCH).transpose(0, 2, 4, 1, 3, 5)
    xp = xp.reshape(_B, _NP, _PD)
    xp = jnp.pad(xp, ((0, 0), (1, _TP - _T), (0, 0)))

    wp = p['patch_w'].reshape(_EMB, -1).T
    posf = p['pos_embed'][0] + jnp.concatenate(
        [p['cls_token'][0],
         jnp.broadcast_to(p['patch_b'][None, :], (_NP, _EMB))], axis=0)
    posf = jnp.pad(posf, ((0, _TP - _T), (0, 0)))
    posf = jnp.tile(posf, (_G, 1))

    blks = p['blocks']
    def stk(key):
        return jnp.stack([b[key] for b in blks])
    ln1w = stk('ln1_w')[:, None, :]
    ln1b = stk('ln1_b')[:, None, :]
    ln2w = stk('ln2_w')[:, None, :]
    ln2b = stk('ln2_b')[:, None, :]
    ipw = stk('in_proj_w')
    ipb = stk('in_proj_b')
    inv_sqrt_dh = 1.0 / math.sqrt(_DH)
    wq = ipw[:, 0 * _EMB:1 * _EMB, :].reshape(_DEPTH, _HEADS, _DH, _EMB)
    wq = wq.transpose(0, 1, 3, 2) * inv_sqrt_dh
    wk = ipw[:, 1 * _EMB:2 * _EMB, :].reshape(_DEPTH, _HEADS, _DH, _EMB)
    wk = wk.transpose(0, 1, 3, 2)
    wv = ipw[:, 2 * _EMB:3 * _EMB, :].reshape(_DEPTH, _HEADS, _DH, _EMB)
    wv = wv.transpose(0, 1, 3, 2)
    wv = jnp.concatenate(
        [wv, jnp.zeros((_DEPTH, _HEADS, _EMB, 1), jnp.float32)], axis=3)
    qb = ipb[:, 0 * _EMB:1 * _EMB].reshape(_DEPTH, _HEADS, 1, _DH) * inv_sqrt_dh
    kb = ipb[:, 1 * _EMB:2 * _EMB].reshape(_DEPTH, _HEADS, 1, _DH)
    vb = ipb[:, 2 * _EMB:3 * _EMB].reshape(_DEPTH, _HEADS, 1, _DH)
    vb = jnp.concatenate(
        [vb, jnp.ones((_DEPTH, _HEADS, 1, 1), jnp.float32)], axis=3)
    wo = stk('out_proj_w').reshape(_DEPTH, _EMB, _HEADS, _DH)
    wo = wo.transpose(0, 2, 3, 1)
    ob = stk('out_proj_b')[:, None, :]
    f1w = jnp.stack([b['ff_w1'].T for b in blks[:3]])
    f1b = jnp.stack([b['ff_b1'] for b in blks[:3]])[:, None, :]
    f2w = jnp.stack([b['ff_w2'].T for b in blks[:3]])
    f2b = jnp.stack([b['ff_b2'] for b in blks[:3]])[:, None, :]
    wr = jnp.stack([b['router_w'] for b in blks[3:]])
    br = jnp.stack([b['router_b'] for b in blks[3:]])[:, :, None]
    nw = p['norm_w'][None, :]
    nb = p['norm_b'][None, :]
    hw = p['head_w'].T
    hb = p['head_b'][None, :]

    nsteps = _B // _G
    out_shapes = (
        jax.ShapeDtypeStruct((_B, 1, _NCLS), jnp.float32),
        jax.ShapeDtypeStruct((nsteps, _NMOE, _E), jnp.float32),
        jax.ShapeDtypeStruct((nsteps, 1, _NMOE), jnp.float32),
    )
    in_specs = [
        pl.BlockSpec((_G, _TP, _PD), lambda b: (b, 0, 0)),
        _full(wp.shape), _full(posf.shape),
        _full(ln1w.shape), _full(ln1b.shape),
        _full(wq.shape), _full(qb.shape), _full(wk.shape), _full(kb.shape),
        _full(wv.shape), _full(vb.shape), _full(wo.shape), _full(ob.shape),
        _full(ln2w.shape), _full(ln2b.shape),
        _full(f1w.shape), _full(f1b.shape), _full(f2w.shape), _full(f2b.shape),
        _full(wr.shape), _full(br.shape),
        _full(nw.shape), _full(nb.shape), _full(hw.shape), _full(hb.shape),
    ]
    out_specs = (
        pl.BlockSpec((_G, 1, _NCLS), lambda b: (b, 0, 0)),
        pl.BlockSpec((1, _NMOE, _E), lambda b: (b, 0, 0)),
        pl.BlockSpec((1, 1, _NMOE), lambda b: (b, 0, 0)),
    )
    logits3, cnt, ssq = pl.pallas_call(
        _vit_body,
        grid=(nsteps,),
        in_specs=in_specs,
        out_specs=out_specs,
        out_shape=out_shapes,
        compiler_params=pltpu.CompilerParams(
            dimension_semantics=("arbitrary",)),
    )(xp, wp, posf, ln1w, ln1b, wq, qb, wk, kb, wv, vb, wo, ob,
      ln2w, ln2b, f1w, f1b, f2w, f2b, wr, br, nw, nb, hw, hb)

    aux = pl.pallas_call(
        _aux_body,
        out_shape=jax.ShapeDtypeStruct((1, 1), jnp.float32),
    )(cnt, ssq)

    return logits3.reshape(_B, _NCLS), aux.reshape(())


# speedup vs baseline: 1.9731x; 1.9731x over previous
import math

import jax
import jax.numpy as jnp
from jax.experimental import pallas as pl
from jax.experimental.pallas import tpu as pltpu

_EMB = 96
_HEADS = 4
_DH = _EMB // _HEADS
_DEPTH = 6
_E = 8
_PATCH = 16
_IMG = 224
_NCLS = 1000
_HID = 4 * _EMB
_B = 128
_NP = (_IMG // _PATCH) ** 2
_T = _NP + 1
_NMOE = _DEPTH - _DEPTH // 2
_CAP = int(1.25 * _B * _T / _E)
_PD = 3 * _PATCH * _PATCH

_TP = 200
_G = 8
_R = _G * _TP


def _ln(v, w, b):
    mu = jnp.mean(v, axis=-1, keepdims=True)
    var = jnp.mean((v - mu) ** 2, axis=-1, keepdims=True)
    return (v - mu) * jax.lax.rsqrt(var + 1e-5) * w + b


def _vit_body(xp_ref, wp_ref, posf_ref,
              ln1w_ref, ln1b_ref, wq_ref, qb_ref, wk_ref, kb_ref,
              wv_ref, vb_ref, wo_ref, ob_ref, ln2w_ref, ln2b_ref,
              f1w_ref, f1b_ref, f2w_ref, f2b_ref, wr_ref, br_ref,
              nw_ref, nb_ref, hw_ref, hb_ref,
              out_ref, cnt_ref, ssq_ref):
    xp = xp_ref[...].reshape(_R, _PD)
    h = jnp.dot(xp, wp_ref[...],
                preferred_element_type=jnp.float32) + posf_ref[...]

    cid = jax.lax.broadcasted_iota(jnp.int32, (1, _R), 1)
    rmaskT = (cid % _TP) < _T
    cmask = jax.lax.broadcasted_iota(jnp.int32, (_TP, _TP), 1) < _T

    cnts = []
    ssqs = []
    for i in range(_DEPTH):
        hn = _ln(h, ln1w_ref[i], ln1b_ref[i])
        attn = jnp.zeros((1, _EMB), jnp.float32) + ob_ref[i]
        for hd in range(_HEADS):
            q = jnp.dot(hn, wq_ref[i, hd],
                        preferred_element_type=jnp.float32) + qb_ref[i, hd]
            k = jnp.dot(hn, wk_ref[i, hd],
                        preferred_element_type=jnp.float32) + kb_ref[i, hd]
            v = jnp.dot(hn, wv_ref[i, hd],
                        preferred_element_type=jnp.float32) + vb_ref[i, hd]
            os_ = []
            for g in range(_G):
                qg = q[g * _TP:(g + 1) * _TP]
                kg = k[g * _TP:(g + 1) * _TP]
                vg = v[g * _TP:(g + 1) * _TP]
                s = jax.lax.dot_general(
                    qg, kg, (((1,), (1,)), ((), ())),
                    preferred_element_type=jnp.float32)
                s = jnp.where(cmask, s, -1e30)
                m = jnp.max(s, axis=1, keepdims=True)
                e = jnp.exp(s - m)
                d = jnp.sum(e, axis=1, keepdims=True)
                og = jnp.dot(e, vg,
                             preferred_element_type=jnp.float32) * (1.0 / d)
                os_.append(og)
            o = jnp.concatenate(os_, axis=0)
            attn = attn + jnp.dot(o, wo_ref[i, hd],
                                  preferred_element_type=jnp.float32)
        h = h + attn
        hn2 = _ln(h, ln2w_ref[i], ln2b_ref[i])
        if i < _DEPTH // 2:
            a = jnp.dot(hn2, f1w_ref[i],
                        preferred_element_type=jnp.float32) + f1b_ref[i]
            f = 0.5 * a * (1.0 + jax.lax.erf(a * (1.0 / math.sqrt(2.0))))
            h = h + jnp.dot(f, f2w_ref[i],
                            preferred_element_type=jnp.float32) + f2b_ref[i]
        else:
            j = i - _DEPTH // 2
            hn2T = hn2.T
            lgT = jnp.dot(wr_ref[j], hn2T,
                          preferred_element_type=jnp.float32) + br_ref[j]
            lgmT = jnp.where(rmaskT, lgT, 0.0)
            ssqs.append(jnp.sum(lgmT * lgmT).reshape(1, 1))
            ids = jax.lax.broadcasted_iota(jnp.int32, (_E, _R), 0)
            m1 = jnp.max(lgT, axis=0, keepdims=True)
            i1 = jnp.min(jnp.where(lgT == m1, ids, _E), axis=0, keepdims=True)
            lg2 = jnp.where(ids == i1, -jnp.inf, lgT)
            m2 = jnp.max(lg2, axis=0, keepdims=True)
            i2 = jnp.min(jnp.where(lg2 == m2, ids, _E), axis=0, keepdims=True)
            pres = jnp.logical_or(ids == i1, ids == i2) & rmaskT
            cnts.append(jnp.sum(pres.astype(jnp.float32), axis=1,
                                keepdims=True))

    cl = jnp.concatenate([h[g * _TP:g * _TP + 1] for g in range(_G)], axis=0)
    cls = _ln(cl, nw_ref[...], nb_ref[...])
    lgt = jnp.dot(cls, hw_ref[...],
                  preferred_element_type=jnp.float32) + hb_ref[...]
    out_ref[...] = lgt.reshape(_G, 1, _NCLS)
    cnt_ref[0] = jnp.concatenate(cnts, axis=1).T
    ssq_ref[0] = jnp.concatenate(ssqs, axis=1)


def _aux_body(cnt_ref, ssq_ref, out_ref):
    c = jnp.sum(cnt_ref[...], axis=0)
    usage = jnp.minimum(c, float(_CAP))
    load = usage / (jnp.sum(usage, axis=1, keepdims=True) + 1e-6)
    bal = float(_E) * jnp.sum(load * load, axis=1)
    ssq = jnp.sum(ssq_ref[...], axis=(0, 1))
    z = 0.001 * ssq / float(_B * _T * _E)
    out_ref[...] = jnp.sum(bal + z).reshape(1, 1)


def _full(shape):
    n = len(shape)
    return pl.BlockSpec(shape, lambda b: (0,) * n)


def kernel(x, params):
    p = params
    hO = _IMG // _PATCH
    xp = x.reshape(_B, 3, hO, _PATCH, hO, _PATCH).transpose(0, 2, 4, 1, 3, 5)
    xp = xp.reshape(_B, _NP, _PD)
    xp = jnp.pad(xp, ((0, 0), (1, _TP - _T), (0, 0)))

    wp = p['patch_w'].reshape(_EMB, -1).T
    posf = p['pos_embed'][0] + jnp.concatenate(
        [p['cls_token'][0],
         jnp.broadcast_to(p['patch_b'][None, :], (_NP, _EMB))], axis=0)
    posf = jnp.pad(posf, ((0, _TP - _T), (0, 0)))
    posf = jnp.tile(posf, (_G, 1))

    blks = p['blocks']
    def stk(key):
        return jnp.stack([b[key] for b in blks])
    ln1w = stk('ln1_w')[:, None, :]
    ln1b = stk('ln1_b')[:, None, :]
    ln2w = stk('ln2_w')[:, None, :]
    ln2b = stk('ln2_b')[:, None, :]
    ipw = stk('in_proj_w')
    ipb = stk('in_proj_b')
    inv_sqrt_dh = 1.0 / math.sqrt(_DH)
    wq = ipw[:, 0 * _EMB:1 * _EMB, :].reshape(_DEPTH, _HEADS, _DH, _EMB)
    wq = wq.transpose(0, 1, 3, 2) * inv_sqrt_dh
    wk = ipw[:, 1 * _EMB:2 * _EMB, :].reshape(_DEPTH, _HEADS, _DH, _EMB)
    wk = wk.transpose(0, 1, 3, 2)
    wv = ipw[:, 2 * _EMB:3 * _EMB, :].reshape(_DEPTH, _HEADS, _DH, _EMB)
    wv = wv.transpose(0, 1, 3, 2)
    qb = ipb[:, 0 * _EMB:1 * _EMB].reshape(_DEPTH, _HEADS, 1, _DH) * inv_sqrt_dh
    kb = ipb[:, 1 * _EMB:2 * _EMB].reshape(_DEPTH, _HEADS, 1, _DH)
    vb = ipb[:, 2 * _EMB:3 * _EMB].reshape(_DEPTH, _HEADS, 1, _DH)
    wo = stk('out_proj_w').reshape(_DEPTH, _EMB, _HEADS, _DH)
    wo = wo.transpose(0, 2, 3, 1)
    ob = stk('out_proj_b')[:, None, :]
    f1w = jnp.stack([b['ff_w1'].T for b in blks[:3]])
    f1b = jnp.stack([b['ff_b1'] for b in blks[:3]])[:, None, :]
    f2w = jnp.stack([b['ff_w2'].T for b in blks[:3]])
    f2b = jnp.stack([b['ff_b2'] for b in blks[:3]])[:, None, :]
    wr = jnp.stack([b['router_w'] for b in blks[3:]])
    br = jnp.stack([b['router_b'] for b in blks[3:]])[:, :, None]
    nw = p['norm_w'][None, :]
    nb = p['norm_b'][None, :]
    hw = p['head_w'].T
    hb = p['head_b'][None, :]

    nsteps = _B // _G
    out_shapes = (
        jax.ShapeDtypeStruct((_B, 1, _NCLS), jnp.float32),
        jax.ShapeDtypeStruct((nsteps, _NMOE, _E), jnp.float32),
        jax.ShapeDtypeStruct((nsteps, 1, _NMOE), jnp.float32),
    )
    in_specs = [
        pl.BlockSpec((_G, _TP, _PD), lambda b: (b, 0, 0)),
        _full(wp.shape), _full(posf.shape),
        _full(ln1w.shape), _full(ln1b.shape),
        _full(wq.shape), _full(qb.shape), _full(wk.shape), _full(kb.shape),
        _full(wv.shape), _full(vb.shape), _full(wo.shape), _full(ob.shape),
        _full(ln2w.shape), _full(ln2b.shape),
        _full(f1w.shape), _full(f1b.shape), _full(f2w.shape), _full(f2b.shape),
        _full(wr.shape), _full(br.shape),
        _full(nw.shape), _full(nb.shape), _full(hw.shape), _full(hb.shape),
    ]
    out_specs = (
        pl.BlockSpec((_G, 1, _NCLS), lambda b: (b, 0, 0)),
        pl.BlockSpec((1, _NMOE, _E), lambda b: (b, 0, 0)),
        pl.BlockSpec((1, 1, _NMOE), lambda b: (b, 0, 0)),
    )
    logits3, cnt, ssq = pl.pallas_call(
        _vit_body,
        grid=(nsteps,),
        in_specs=in_specs,
        out_specs=out_specs,
        out_shape=out_shapes,
        compiler_params=pltpu.CompilerParams(
            dimension_semantics=("arbitrary",)),
    )(xp, wp, posf, ln1w, ln1b, wq, qb, wk, kb, wv, vb, wo, ob,
      ln2w, ln2b, f1w, f1b, f2w, f2b, wr, br, nw, nb, hw, hb)

    aux = pl.pallas_call(
        _aux_body,
        out_shape=jax.ShapeDtypeStruct((1, 1), jnp.float32),
    )(cnt, ssq)

    return logits3.reshape(_B, _NCLS), aux.reshape(())
